# Initial kernel scaffold; baseline (speedup 1.0000x reference)
import jax
import jax.numpy as jnp
from jax.experimental import pallas as pl


def kernel(x, W_enc, b_enc, W_attn, w_attn, W_cls, b_cls):
    raise NotImplementedError("write your pallas kernel here")


# speedup vs baseline: 1.2160x; 1.2160x over previous
import functools

import jax
import jax.numpy as jnp
from jax.experimental import pallas as pl
from jax.experimental.pallas import tpu as pltpu

N_PATCHES = 8192
INPUT_DIM = 1024
ATTN_DIM = 384
TOP_K = 512
GRID = 8
TILE = N_PATCHES // GRID


def _count_ge(key, thresh):
    return jnp.sum((key >= thresh).astype(jnp.int32))


def _fused_kernel(x_ref, Wenc_ref, benc_ref, Wattn_ref, wattn_ref,
                  Wcls_ref, bcls_ref, out_ref, feats_ref, scores_ref):
    i = pl.program_id(0)

    f = jnp.dot(x_ref[...], Wenc_ref[...], preferred_element_type=jnp.float32)
    f = jnp.maximum(f + benc_ref[...], 0.0)
    feats_ref[pl.ds(i * TILE, TILE), :] = f

    t = jnp.tanh(jnp.dot(f, Wattn_ref[...], preferred_element_type=jnp.float32))
    s = jnp.dot(t, wattn_ref[...], preferred_element_type=jnp.float32)

    lane = jax.lax.broadcasted_iota(jnp.int32, (TILE, GRID), 1)
    scores_ref[...] = jnp.where(lane == i, s, scores_ref[...])

    @pl.when(i == GRID - 1)
    def _finalize():
        scores = scores_ref[...]

        bits = jax.lax.bitcast_convert_type(scores, jnp.int32)
        key = jnp.where(bits >= 0, bits, bits ^ jnp.int32(0x7FFFFFFF))

        nonneg = _count_ge(key, jnp.int32(0)) >= TOP_K
        kth0 = jnp.where(nonneg, jnp.int32(0), jnp.int32(-0x80000000))

        def _body(b, kth):
            cand = kth | (jnp.int32(1) << (30 - b))
            return jnp.where(_count_ge(key, cand) >= TOP_K, cand, kth)

        kth = jax.lax.fori_loop(0, 31, _body, kth0)

        n_gt = _count_ge(key, kth + 1)
        take_ties = TOP_K - n_gt
        tie = (key == kth)
        row = jax.lax.broadcasted_iota(jnp.int32, (TILE, GRID), 0)
        col = jax.lax.broadcasted_iota(jnp.int32, (TILE, GRID), 1)
        flat = col * TILE + row
        tie_i = tie.astype(jnp.int32)

        def _jbody(b, J):
            cand = J | (jnp.int32(1) << (13 - b))
            cnt = jnp.sum(jnp.where(flat < cand, tie_i, 0))
            return jnp.where(cnt <= take_ties, cand, J)

        J = jax.lax.fori_loop(0, 14, _jbody, jnp.int32(0))
        sel = (key > kth) | (tie & (flat < J))

        m = jnp.max(scores)
        w = jnp.where(sel, jnp.exp(scores - m), 0.0)
        w = w / jnp.sum(w)

        dn = (((0,), (0,)), ((), ()))
        bag = jnp.zeros((1, INPUT_DIM), dtype=jnp.float32)
        for c in range(GRID):
            bag = bag + jax.lax.dot_general(
                w[:, c:c + 1], feats_ref[pl.ds(c * TILE, TILE), :], dn,
                preferred_element_type=jnp.float32)

        logits = jnp.dot(bag, Wcls_ref[...], preferred_element_type=jnp.float32)
        out_ref[...] = logits + bcls_ref[...]


@jax.jit
def kernel(x, W_enc, b_enc, W_attn, w_attn, W_cls, b_cls):
    ncls = W_cls.shape[1]
    Wcls_p = jnp.zeros((INPUT_DIM, 128), jnp.float32).at[:, :ncls].set(W_cls)
    bcls_p = jnp.zeros((1, 128), jnp.float32).at[0, :ncls].set(b_cls)

    out = pl.pallas_call(
        _fused_kernel,
        grid=(GRID,),
        in_specs=[
            pl.BlockSpec((TILE, INPUT_DIM), lambda i: (i, 0)),
            pl.BlockSpec((INPUT_DIM, INPUT_DIM), lambda i: (0, 0)),
            pl.BlockSpec((1, INPUT_DIM), lambda i: (0, 0)),
            pl.BlockSpec((INPUT_DIM, ATTN_DIM), lambda i: (0, 0)),
            pl.BlockSpec((ATTN_DIM, 1), lambda i: (0, 0)),
            pl.BlockSpec((INPUT_DIM, 128), lambda i: (0, 0)),
            pl.BlockSpec((1, 128), lambda i: (0, 0)),
        ],
        out_specs=pl.BlockSpec((1, 128), lambda i: (0, 0)),
        out_shape=jax.ShapeDtypeStruct((1, 128), jnp.float32),
        scratch_shapes=[
            pltpu.VMEM((N_PATCHES, INPUT_DIM), jnp.float32),
            pltpu.VMEM((TILE, GRID), jnp.float32),
        ],
    )(x, W_enc, b_enc.reshape(1, INPUT_DIM), W_attn,
      w_attn.reshape(ATTN_DIM, 1), Wcls_p, bcls_p)
    return out[:, :ncls]


# speedup vs baseline: 1.3321x; 1.0955x over previous
import functools

import jax
import jax.numpy as jnp
from jax.experimental import pallas as pl
from jax.experimental.pallas import tpu as pltpu

N_PATCHES = 8192
INPUT_DIM = 1024
ATTN_DIM = 384
TOP_K = 512
GRID = 8
TILE = N_PATCHES // GRID


def _count_ge(key, thresh):
    return jnp.sum((key >= thresh).astype(jnp.int32))


def _fused_kernel(x_ref, Wenc_ref, benc_ref, Wattn_ref, wattn_ref,
                  Wcls_ref, bcls_ref, out_ref, feats_ref, scores_ref):
    i = pl.program_id(0)

    f = jnp.dot(x_ref[...], Wenc_ref[...], preferred_element_type=jnp.float32)
    f = jnp.maximum(f + benc_ref[...], 0.0)
    feats_ref[pl.ds(i * TILE, TILE), :] = f

    t = jnp.tanh(jnp.dot(f, Wattn_ref[...], preferred_element_type=jnp.float32))
    s = jnp.dot(t, wattn_ref[...], preferred_element_type=jnp.float32)

    lane = jax.lax.broadcasted_iota(jnp.int32, (TILE, GRID), 1)
    scores_ref[...] = jnp.where(lane == i, s, scores_ref[...])

    @pl.when(i == GRID - 1)
    def _finalize():
        scores = scores_ref[...].T

        bits = jax.lax.bitcast_convert_type(scores, jnp.int32)
        key = jnp.where(bits >= 0, bits, bits ^ jnp.int32(0x7FFFFFFF))

        nonneg = _count_ge(key, jnp.int32(0)) >= TOP_K
        kth0 = jnp.where(nonneg, jnp.int32(0), jnp.int32(-0x80000000))

        def _body(b, kth):
            cand = kth | (jnp.int32(1) << (30 - b))
            return jnp.where(_count_ge(key, cand) >= TOP_K, cand, kth)

        kth = jax.lax.fori_loop(0, 31, _body, kth0)

        n_gt = _count_ge(key, kth + 1)
        take_ties = TOP_K - n_gt
        tie = (key == kth)
        trow = jax.lax.broadcasted_iota(jnp.int32, (GRID, TILE), 0)
        tcol = jax.lax.broadcasted_iota(jnp.int32, (GRID, TILE), 1)
        flat = trow * TILE + tcol
        tie_i = tie.astype(jnp.int32)

        def _jbody(b, J):
            cand = J | (jnp.int32(1) << (13 - b))
            cnt = jnp.sum(jnp.where(flat < cand, tie_i, 0))
            return jnp.where(cnt <= take_ties, cand, J)

        J = jax.lax.fori_loop(0, 14, _jbody, jnp.int32(0))
        sel = (key > kth) | (tie & (flat < J))

        m = jnp.max(scores)
        w = jnp.where(sel, jnp.exp(scores - m), 0.0)
        w = (w / jnp.sum(w)).T

        dn = (((0,), (0,)), ((), ()))
        bag = jnp.zeros((1, INPUT_DIM), dtype=jnp.float32)
        for c in range(GRID):
            bag = bag + jax.lax.dot_general(
                w[:, c:c + 1], feats_ref[pl.ds(c * TILE, TILE), :], dn,
                preferred_element_type=jnp.float32)

        logits = jnp.dot(bag, Wcls_ref[...], preferred_element_type=jnp.float32)
        out_ref[...] = logits + bcls_ref[...]


@jax.jit
def kernel(x, W_enc, b_enc, W_attn, w_attn, W_cls, b_cls):
    ncls = W_cls.shape[1]
    Wcls_p = jnp.zeros((INPUT_DIM, 128), jnp.float32).at[:, :ncls].set(W_cls)
    bcls_p = jnp.zeros((1, 128), jnp.float32).at[0, :ncls].set(b_cls)

    out = pl.pallas_call(
        _fused_kernel,
        grid=(GRID,),
        in_specs=[
            pl.BlockSpec((TILE, INPUT_DIM), lambda i: (i, 0)),
            pl.BlockSpec((INPUT_DIM, INPUT_DIM), lambda i: (0, 0)),
            pl.BlockSpec((1, INPUT_DIM), lambda i: (0, 0)),
            pl.BlockSpec((INPUT_DIM, ATTN_DIM), lambda i: (0, 0)),
            pl.BlockSpec((ATTN_DIM, 1), lambda i: (0, 0)),
            pl.BlockSpec((INPUT_DIM, 128), lambda i: (0, 0)),
            pl.BlockSpec((1, 128), lambda i: (0, 0)),
        ],
        out_specs=pl.BlockSpec((1, 128), lambda i: (0, 0)),
        out_shape=jax.ShapeDtypeStruct((1, 128), jnp.float32),
        scratch_shapes=[
            pltpu.VMEM((N_PATCHES, INPUT_DIM), jnp.float32),
            pltpu.VMEM((TILE, GRID), jnp.float32),
        ],
    )(x, W_enc, b_enc.reshape(1, INPUT_DIM), W_attn,
      w_attn.reshape(ATTN_DIM, 1), Wcls_p, bcls_p)
    return out[:, :ncls]
